# baseline (speedup 1.0000x reference)
import jax
import jax.numpy as jnp
from jax import lax
from jax.experimental import pallas as pl
from jax.experimental.pallas import tpu as pltpu
from jax.experimental.pallas import tpu_sc as plsc

_VOCAB = 1000000
_H = 64
_CTX = 200
_NCORES = 2
_NSUB = 16
_IDS_PER_W = 8
_ACTIVE_W = 25
_PACK = 2 * _H
_ROWS2 = _VOCAB // 2
_MV_BLOCK = 16384


def _sc_gather_body(ids_hbm, table_hbm, out_hbm, idx_v, rows_v, sem):
    wid = lax.axis_index("s") * _NCORES + lax.axis_index("c")

    @pl.when(wid < _ACTIVE_W)
    def _():
        base = wid * _IDS_PER_W
        pltpu.sync_copy(ids_hbm.at[pl.ds(base, _IDS_PER_W)], idx_v)
        pltpu.async_copy(table_hbm.at[idx_v], rows_v, sem).wait()
        pltpu.sync_copy(rows_v, out_hbm.at[pl.ds(base, _IDS_PER_W)])


def _sc_gather(ids_half, table2):
    return pl.kernel(
        _sc_gather_body,
        out_type=jax.ShapeDtypeStruct((_CTX, _PACK), jnp.float32),
        mesh=plsc.VectorSubcoreMesh(
            core_axis_name="c", subcore_axis_name="s",
            num_cores=_NCORES, num_subcores=_NSUB),
        scratch_types=[
            pltpu.VMEM((_IDS_PER_W,), jnp.int32),
            pltpu.VMEM((_IDS_PER_W, _PACK), jnp.float32),
            pltpu.SemaphoreType.DMA,
        ],
    )(ids_half, table2)


def _head_body(nidx_ref, pidx_ref, lidx_ref, rows_ref, ids_ref, node_ref,
               par_ref, lang_ref, nf_ref, numw_ref, numb_ref, hidw_ref,
               hidb_ref, sww_ref, swb_ref, hh2_out, sw_out):
    even = (ids_ref[...] % 2) == 0
    sel = jnp.where(even, rows_ref[:, :_H], rows_ref[:, _H:])
    tok = jnp.sum(sel, axis=0, keepdims=True) * (1.0 / _CTX)
    ni = nidx_ref[0]
    pi = pidx_ref[0]
    li = lidx_ref[0]
    feat = (node_ref[pl.ds(ni, 1), :]
            + par_ref[pl.ds(pi, 1), :]
            + lang_ref[pl.ds(li, 1), :])
    nproj = lax.dot_general(nf_ref[...], numw_ref[...], (((1,), (1,)), ((), ())),
                            preferred_element_type=jnp.float32)
    feat = feat + nproj + numb_ref[...]
    cat = jnp.concatenate([tok, feat], axis=1)
    hid = jnp.tanh(
        lax.dot_general(cat, hidw_ref[...], (((1,), (1,)), ((), ())),
                        preferred_element_type=jnp.float32)
        + hidb_ref[...])
    sw_out[...] = jnp.sum(hid * sww_ref[...], axis=1, keepdims=True) + swb_ref[0]
    zero = jnp.zeros((1, _H), jnp.float32)
    hh2_out[...] = jnp.concatenate(
        [jnp.concatenate([hid, zero], axis=1),
         jnp.concatenate([zero, hid], axis=1)], axis=0)


def _head(nidx, pidx, lidx, rows, ids, node_table, parent_table, lang_table,
          nf, num_w, num_b, hid_w, hid_b, sw_w, sw_b):
    smem = pl.BlockSpec(memory_space=pltpu.SMEM)
    vmem = pl.BlockSpec(memory_space=pltpu.VMEM)
    return pl.pallas_call(
        _head_body,
        in_specs=[smem, smem, smem] + [vmem] * 11 + [smem],
        out_shape=(jax.ShapeDtypeStruct((2, _PACK), jnp.float32),
                   jax.ShapeDtypeStruct((1, 1), jnp.float32)),
    )(nidx, pidx, lidx, rows, ids, node_table, parent_table, lang_table,
      nf, num_w, num_b, hid_w, hid_b, sw_w, sw_b)


def _mv_body(hh2_ref, w_ref, b_ref, o_ref):
    res = lax.dot_general(hh2_ref[...], w_ref[...], (((1,), (1,)), ((), ())),
                          preferred_element_type=jnp.float32)
    o_ref[...] = res + b_ref[...]


def _matvec(hh2, pref_w2, b2):
    grid = pl.cdiv(_ROWS2, _MV_BLOCK)
    return pl.pallas_call(
        _mv_body,
        grid=(grid,),
        in_specs=[
            pl.BlockSpec((2, _PACK), lambda i: (0, 0)),
            pl.BlockSpec((_MV_BLOCK, _PACK), lambda i: (i, 0)),
            pl.BlockSpec((2, _MV_BLOCK), lambda i: (0, i)),
        ],
        out_specs=pl.BlockSpec((2, _MV_BLOCK), lambda i: (0, i)),
        out_shape=jax.ShapeDtypeStruct((2, _ROWS2), jnp.float32),
    )(hh2, pref_w2, b2)


def kernel(prefix_ids, node_idx, parent_idx, lang_idx, numeric_features,
           token_table, node_table, parent_table, lang_table,
           num_W, num_b, hid_W, hid_b, sw_W, sw_b, pref_W, pref_b):
    ids = prefix_ids[-_CTX:].astype(jnp.int32)
    table2 = token_table.reshape(_ROWS2, _PACK)
    pref_w2 = pref_W.reshape(_ROWS2, _PACK)
    rows = _sc_gather(ids // 2, table2)
    nidx = jnp.asarray(node_idx, jnp.int32).reshape(1)
    pidx = jnp.asarray(parent_idx, jnp.int32).reshape(1)
    lidx = jnp.asarray(lang_idx, jnp.int32).reshape(1)
    hh2, sw = _head(
        nidx, pidx, lidx, rows, ids.reshape(_CTX, 1), node_table,
        parent_table, lang_table, numeric_features.reshape(1, 3), num_W,
        num_b.reshape(1, _H), hid_W, hid_b.reshape(1, _H), sw_W,
        sw_b.reshape(1))
    b2 = jnp.stack([pref_b[0::2], pref_b[1::2]], axis=0)
    out2 = _matvec(hh2, pref_w2, b2)
    logits = jnp.stack([out2[0], out2[1]], axis=-1).reshape(_VOCAB)
    return sw[0, 0], logits


# speedup vs baseline: 1.9738x; 1.9738x over previous
import jax
import jax.numpy as jnp
from jax import lax
from jax.experimental import pallas as pl
from jax.experimental.pallas import tpu as pltpu

_VOCAB = 1000000
_H = 64
_CTX = 200
_DEPTH = 8
_MV_BLOCK = 32768
_MV_GRID = pl.cdiv(_VOCAB, _MV_BLOCK)
_MV_LAST = _MV_GRID - 1
_MV_TAIL = _VOCAB - _MV_LAST * _MV_BLOCK


def _head_body(ids_ref, nidx_ref, pidx_ref, lidx_ref, swb_ref, tok_hbm,
               node_hbm, par_hbm, lang_hbm, nf_ref, numw_ref, numb_ref,
               hidw_ref, hidb_ref, sww_ref, hid_out, sw_out,
               buf, fbuf, sems, fsems):
    pltpu.make_async_copy(node_hbm.at[pl.ds(nidx_ref[0], 1), :],
                          fbuf.at[pl.ds(0, 1), :], fsems.at[0]).start()
    pltpu.make_async_copy(par_hbm.at[pl.ds(pidx_ref[0], 1), :],
                          fbuf.at[pl.ds(1, 1), :], fsems.at[1]).start()
    pltpu.make_async_copy(lang_hbm.at[pl.ds(lidx_ref[0], 1), :],
                          fbuf.at[pl.ds(2, 1), :], fsems.at[2]).start()

    def _issue(j, slot):
        pltpu.make_async_copy(tok_hbm.at[pl.ds(ids_ref[j], 1), :],
                              buf.at[pl.ds(slot, 1), :], sems.at[slot]).start()

    for k in range(_DEPTH):
        _issue(k, k)

    def _step(j, acc):
        slot = lax.rem(j, _DEPTH)
        pltpu.make_async_copy(tok_hbm.at[pl.ds(ids_ref[j], 1), :],
                              buf.at[pl.ds(slot, 1), :], sems.at[slot]).wait()
        acc = acc + buf[pl.ds(slot, 1), :]

        @pl.when(j + _DEPTH < _CTX)
        def _():
            _issue(j + _DEPTH, slot)

        return acc

    acc = lax.fori_loop(0, _CTX, _step, jnp.zeros((1, _H), jnp.float32))
    tok = acc * (1.0 / _CTX)

    pltpu.make_async_copy(node_hbm.at[pl.ds(nidx_ref[0], 1), :],
                          fbuf.at[pl.ds(0, 1), :], fsems.at[0]).wait()
    pltpu.make_async_copy(par_hbm.at[pl.ds(pidx_ref[0], 1), :],
                          fbuf.at[pl.ds(1, 1), :], fsems.at[1]).wait()
    pltpu.make_async_copy(lang_hbm.at[pl.ds(lidx_ref[0], 1), :],
                          fbuf.at[pl.ds(2, 1), :], fsems.at[2]).wait()

    nproj = lax.dot_general(nf_ref[...], numw_ref[...], (((1,), (1,)), ((), ())),
                            preferred_element_type=jnp.float32)
    feat = (fbuf[pl.ds(0, 1), :] + fbuf[pl.ds(1, 1), :] + fbuf[pl.ds(2, 1), :]
            + nproj + numb_ref[...])
    cat = jnp.concatenate([tok, feat], axis=1)
    hid = jnp.tanh(
        lax.dot_general(cat, hidw_ref[...], (((1,), (1,)), ((), ())),
                        preferred_element_type=jnp.float32)
        + hidb_ref[...])
    hid_out[...] = hid
    sw_out[...] = jnp.sum(hid * sww_ref[...], axis=1, keepdims=True) + swb_ref[0]


def _head(ids, nidx, pidx, lidx, token_table, node_table, parent_table,
          lang_table, nf, num_w, num_b, hid_w, hid_b, sw_w, sw_b):
    smem = pl.BlockSpec(memory_space=pltpu.SMEM)
    vmem = pl.BlockSpec(memory_space=pltpu.VMEM)
    hbm = pl.BlockSpec(memory_space=pltpu.MemorySpace.HBM)
    return pl.pallas_call(
        _head_body,
        in_specs=[smem, smem, smem, smem, smem, hbm, hbm, hbm, hbm,
                  vmem, vmem, vmem, vmem, vmem, vmem],
        out_shape=(jax.ShapeDtypeStruct((1, _H), jnp.float32),
                   jax.ShapeDtypeStruct((1, 1), jnp.float32)),
        scratch_shapes=[
            pltpu.VMEM((_DEPTH, _H), jnp.float32),
            pltpu.VMEM((4, _H), jnp.float32),
            pltpu.SemaphoreType.DMA((_DEPTH,)),
            pltpu.SemaphoreType.DMA((4,)),
        ],
    )(ids, nidx, pidx, lidx, sw_b, token_table, node_table, parent_table,
      lang_table, nf, num_w, num_b, hid_w, hid_b, sw_w)


def _mv_body(h_ref, b_ref, w_hbm, o_ref, bufs, sems):
    i = pl.program_id(0)

    def _issue(bi, slot):
        @pl.when(bi < _MV_LAST)
        def _():
            pltpu.make_async_copy(
                w_hbm.at[pl.ds(bi * _MV_BLOCK, _MV_BLOCK), :],
                bufs.at[slot], sems.at[slot]).start()

        @pl.when(bi == _MV_LAST)
        def _():
            pltpu.make_async_copy(
                w_hbm.at[pl.ds(bi * _MV_BLOCK, _MV_TAIL), :],
                bufs.at[slot, pl.ds(0, _MV_TAIL), :], sems.at[slot]).start()

    @pl.when(i == 0)
    def _():
        _issue(0, 0)

    @pl.when(i + 1 < _MV_GRID)
    def _():
        _issue(i + 1, (i + 1) % 2)

    slot = lax.rem(i, 2)

    @pl.when(i < _MV_LAST)
    def _():
        pltpu.make_async_copy(w_hbm.at[pl.ds(i * _MV_BLOCK, _MV_BLOCK), :],
                              bufs.at[slot], sems.at[slot]).wait()

    @pl.when(i == _MV_LAST)
    def _():
        pltpu.make_async_copy(w_hbm.at[pl.ds(i * _MV_BLOCK, _MV_TAIL), :],
                              bufs.at[slot, pl.ds(0, _MV_TAIL), :],
                              sems.at[slot]).wait()
    w = bufs[slot]
    res = lax.dot_general(h_ref[...], w, (((1,), (1,)), ((), ())),
                          preferred_element_type=jnp.float32)
    o_ref[...] = res[0, :] + b_ref[...]


def _matvec(hidden, pref_w, pref_b):
    return pl.pallas_call(
        _mv_body,
        grid=(_MV_GRID,),
        in_specs=[
            pl.BlockSpec((1, _H), lambda i: (0, 0)),
            pl.BlockSpec((_MV_BLOCK,), lambda i: (i,)),
            pl.BlockSpec(memory_space=pltpu.MemorySpace.HBM),
        ],
        out_specs=pl.BlockSpec((_MV_BLOCK,), lambda i: (i,)),
        out_shape=jax.ShapeDtypeStruct((_VOCAB,), jnp.float32),
        scratch_shapes=[
            pltpu.VMEM((2, _MV_BLOCK, _H), jnp.float32),
            pltpu.SemaphoreType.DMA((2,)),
        ],
    )(hidden, pref_b, pref_w)


def kernel(prefix_ids, node_idx, parent_idx, lang_idx, numeric_features,
           token_table, node_table, parent_table, lang_table,
           num_W, num_b, hid_W, hid_b, sw_W, sw_b, pref_W, pref_b):
    ids = prefix_ids[-_CTX:].astype(jnp.int32)
    nidx = jnp.asarray(node_idx, jnp.int32).reshape(1)
    pidx = jnp.asarray(parent_idx, jnp.int32).reshape(1)
    lidx = jnp.asarray(lang_idx, jnp.int32).reshape(1)
    hidden, sw = _head(
        ids, nidx, pidx, lidx, token_table, node_table, parent_table,
        lang_table, numeric_features.reshape(1, 3), num_W,
        num_b.reshape(1, _H), hid_W, hid_b.reshape(1, _H), sw_W,
        sw_b.reshape(1))
    logits = _matvec(hidden, pref_W, pref_b)
    return sw[0, 0], logits
